# Initial kernel scaffold; baseline (speedup 1.0000x reference)
import jax
import jax.numpy as jnp
from jax.experimental import pallas as pl


def kernel(x, edge_index, edge_attr, batch, be_W1, be_b1, be_g1, be_bb1, be_W2, be_b2, be_g2, be_bb2, mlp_W1, mlp_b1, mlp_g1, mlp_bb1, mlp_W2, mlp_b2, mlp_g2, mlp_bb2, eps, W_ih, W_hh, b_ih, b_hh, fc1_W, fc1_b, fc4_W, fc4_b):
    raise NotImplementedError("write your pallas kernel here")


# speedup vs baseline: 1.6249x; 1.6249x over previous
import functools

import jax
import jax.numpy as jnp
from jax import lax
from jax.experimental import pallas as pl
from jax.experimental.pallas import tpu as pltpu
from jax.experimental.pallas import tpu_sc as plsc

N = 10000
E = 160000
B = 64
DIM = 128
EDIM = 16
L = 6
NC = 12

EBLK = 2000
NEB = E // EBLK

SC_CORES = 2
SC_SUB = 16
NW = SC_CORES * SC_SUB
EPW = E // NW
CHUNK = 40
NCHUNK = EPW // CHUNK
N_PAD = 10240
ROWS_PER_SUB = N_PAD // SC_SUB
ZROWS = 128


def _edge_stats1_body(ea_ref, w1t_ref, s_ref, q_ref):
    i = pl.program_id(0)

    @pl.when(i == 0)
    def _():
        s_ref[...] = jnp.zeros_like(s_ref)
        q_ref[...] = jnp.zeros_like(q_ref)

    blk = ea_ref[...]
    for l in range(L):
        y = jnp.dot(blk, w1t_ref[l], preferred_element_type=jnp.float32)
        s_ref[l : l + 1, :] += jnp.sum(y, axis=0, keepdims=True)
        q_ref[l : l + 1, :] += jnp.sum(y * y, axis=0, keepdims=True)


def _edge_stats1(edge_attr, w1t_all):
    return pl.pallas_call(
        _edge_stats1_body,
        grid=(NEB,),
        in_specs=[
            pl.BlockSpec((EBLK, EDIM), lambda i: (i, 0)),
            pl.BlockSpec((L, EDIM, DIM), lambda i: (0, 0, 0)),
        ],
        out_specs=[
            pl.BlockSpec((L, DIM), lambda i: (0, 0)),
            pl.BlockSpec((L, DIM), lambda i: (0, 0)),
        ],
        out_shape=[
            jax.ShapeDtypeStruct((L, DIM), jnp.float32),
            jax.ShapeDtypeStruct((L, DIM), jnp.float32),
        ],
    )(edge_attr, w1t_all)


def _edge_mlp_body(ea_ref, w1t_ref, a1_ref, c1_ref, w2t_ref, z_ref, s_ref, q_ref):
    i = pl.program_id(0)

    @pl.when(i == 0)
    def _():
        s_ref[...] = jnp.zeros_like(s_ref)
        q_ref[...] = jnp.zeros_like(q_ref)

    y = jnp.dot(ea_ref[...], w1t_ref[...], preferred_element_type=jnp.float32)
    u = jnp.maximum(y * a1_ref[...] + c1_ref[...], 0.0)
    z = jnp.dot(u, w2t_ref[...], preferred_element_type=jnp.float32)
    z_ref[...] = z
    s_ref[...] += jnp.sum(z, axis=0, keepdims=True)
    q_ref[...] += jnp.sum(z * z, axis=0, keepdims=True)


def _edge_mlp(edge_attr, w1t, a1, c1, w2t):
    return pl.pallas_call(
        _edge_mlp_body,
        grid=(NEB,),
        in_specs=[
            pl.BlockSpec((EBLK, EDIM), lambda i: (i, 0)),
            pl.BlockSpec((EDIM, DIM), lambda i: (0, 0)),
            pl.BlockSpec((1, DIM), lambda i: (0, 0)),
            pl.BlockSpec((1, DIM), lambda i: (0, 0)),
            pl.BlockSpec((DIM, DIM), lambda i: (0, 0)),
        ],
        out_specs=[
            pl.BlockSpec((EBLK, DIM), lambda i: (i, 0)),
            pl.BlockSpec((1, DIM), lambda i: (0, 0)),
            pl.BlockSpec((1, DIM), lambda i: (0, 0)),
        ],
        out_shape=[
            jax.ShapeDtypeStruct((E, DIM), jnp.float32),
            jax.ShapeDtypeStruct((1, DIM), jnp.float32),
            jax.ShapeDtypeStruct((1, DIM), jnp.float32),
        ],
    )(edge_attr, w1t, a1, c1, w2t)


def _sc_msg_body(h_hbm, z2_hbm, src_hbm, dst_hbm, ac_hbm, out_hbm,
                 acc, zbuf, srcv, dstv, hrows, zrows, msg, acv, sem):
    cid = lax.axis_index("c")
    sid = lax.axis_index("s")
    wid = cid * SC_SUB + sid

    pltpu.sync_copy(ac_hbm, acv)

    @pl.loop(0, ZROWS)
    def _(r):
        for q in range(DIM // 16):
            zbuf[r, pl.ds(q * 16, 16)] = jnp.zeros((16,), jnp.float32)

    @pl.loop(0, ROWS_PER_SUB // ZROWS)
    def _(j):
        pltpu.sync_copy(zbuf, acc.at[pl.ds(sid * ROWS_PER_SUB + j * ZROWS, ZROWS)])

    plsc.subcore_barrier()

    @pl.loop(0, NCHUNK)
    def _(ch):
        base = wid * EPW + ch * CHUNK
        pltpu.sync_copy(src_hbm.at[pl.ds(base, CHUNK)], srcv)
        pltpu.sync_copy(dst_hbm.at[pl.ds(base, CHUNK)], dstv)
        cp = pltpu.async_copy(h_hbm.at[srcv], hrows, sem)
        pltpu.sync_copy(z2_hbm.at[pl.ds(base, CHUNK)], zrows)
        cp.wait()

        @pl.loop(0, CHUNK)
        def _(j):
            for q in range(DIM // 16):
                sl = pl.ds(q * 16, 16)
                e = jnp.maximum(zrows[j, sl] * acv[0, sl] + acv[1, sl], 0.0)
                msg[j, sl] = jnp.maximum(hrows[j, sl] + e, 0.0)

        pltpu.sync_copy(msg, acc.at[dstv], add=True)

    plsc.subcore_barrier()

    pltpu.sync_copy(
        acc.at[pl.ds(sid * ROWS_PER_SUB, ROWS_PER_SUB)],
        out_hbm.at[cid, pl.ds(sid * ROWS_PER_SUB, ROWS_PER_SUB)],
    )


@functools.lru_cache(maxsize=1)
def _build_sc_msg():
    return pl.kernel(
        _sc_msg_body,
        out_type=jax.ShapeDtypeStruct((SC_CORES, N_PAD, DIM), jnp.float32),
        mesh=plsc.VectorSubcoreMesh(
            core_axis_name="c", subcore_axis_name="s",
            num_cores=SC_CORES, num_subcores=SC_SUB,
        ),
        scratch_types=[
            pltpu.MemorySpace.VMEM_SHARED((N_PAD, DIM), jnp.float32),
            pltpu.VMEM((ZROWS, DIM), jnp.float32),
            pltpu.VMEM((CHUNK,), jnp.int32),
            pltpu.VMEM((CHUNK,), jnp.int32),
            pltpu.VMEM((CHUNK, DIM), jnp.float32),
            pltpu.VMEM((CHUNK, DIM), jnp.float32),
            pltpu.VMEM((CHUNK, DIM), jnp.float32),
            pltpu.VMEM((2, DIM), jnp.float32),
            pltpu.SemaphoreType.DMA,
        ],
    )


def _sc_msg(h, z2, src, dst, ac):
    return _build_sc_msg()(h, z2, src, dst, ac)


def _node_mlp_body(h_ref, agg_ref, eps_ref, w1t_ref, g1_ref, bb1_ref,
                   w2t_ref, g2_ref, bb2_ref, out_ref):
    z = (1.0 + eps_ref[0, 0]) * h_ref[...] + agg_ref[0, :N, :] + agg_ref[1, :N, :]
    y = jnp.dot(z, w1t_ref[...], preferred_element_type=jnp.float32)
    m = jnp.mean(y, axis=0, keepdims=True)
    yc = y - m
    v = jnp.mean(yc * yc, axis=0, keepdims=True)
    u = jnp.maximum(yc * jax.lax.rsqrt(v + 1e-5) * g1_ref[...] + bb1_ref[...], 0.0)
    y2 = jnp.dot(u, w2t_ref[...], preferred_element_type=jnp.float32)
    m2 = jnp.mean(y2, axis=0, keepdims=True)
    yc2 = y2 - m2
    v2 = jnp.mean(yc2 * yc2, axis=0, keepdims=True)
    out_ref[...] = jnp.maximum(
        yc2 * jax.lax.rsqrt(v2 + 1e-5) * g2_ref[...] + bb2_ref[...], 0.0)


def _node_mlp(h, agg, eps_l, w1t, g1, bb1, w2t, g2, bb2):
    return pl.pallas_call(
        _node_mlp_body,
        out_shape=jax.ShapeDtypeStruct((N, DIM), jnp.float32),
    )(h, agg, eps_l, w1t, g1, bb1, w2t, g2, bb2)


def _readout_body(h_ref, b_ref, wiht_ref, whht_ref, bih_ref, bhh_ref,
                  fc1t_ref, fc1b_ref, fc4t_ref, fc4b_ref, out_ref):
    h = h_ref[...]
    bcol = b_ref[...]
    mask = bcol == lax.broadcasted_iota(jnp.int32, (N, B), 1)
    maskf = mask.astype(jnp.float32)

    q_star = jnp.zeros((B, 2 * DIM), jnp.float32)
    hh = jnp.zeros((B, DIM), jnp.float32)
    cc = jnp.zeros((B, DIM), jnp.float32)
    for _ in range(6):
        gates = (jnp.dot(q_star, wiht_ref[...], preferred_element_type=jnp.float32)
                 + bih_ref[...]
                 + jnp.dot(hh, whht_ref[...], preferred_element_type=jnp.float32)
                 + bhh_ref[...])
        gi = jax.nn.sigmoid(gates[:, :DIM])
        gf = jax.nn.sigmoid(gates[:, DIM:2 * DIM])
        gg = jnp.tanh(gates[:, 2 * DIM:3 * DIM])
        go = jax.nn.sigmoid(gates[:, 3 * DIM:])
        cc = gf * cc + gi * gg
        hh = go * jnp.tanh(cc)
        q = hh
        ee_all = lax.dot_general(h, q, (((1,), (1,)), ((), ())),
                                 preferred_element_type=jnp.float32)
        mmax = jnp.max(jnp.where(mask, ee_all, -1e30), axis=0, keepdims=True)
        ee = jnp.sum(jnp.where(mask, ee_all, 0.0), axis=1, keepdims=True)
        mnode = jnp.sum(jnp.where(mask, mmax, 0.0), axis=1, keepdims=True)
        a_un = jnp.exp(ee - mnode)
        denom = jnp.sum(a_un * maskf, axis=0, keepdims=True)
        dnode = jnp.sum(jnp.where(mask, denom, 0.0), axis=1, keepdims=True)
        a = a_un / (dnode + 1e-16)
        attn = maskf * a
        r = lax.dot_general(attn, h, (((0,), (0,)), ((), ())),
                            preferred_element_type=jnp.float32)
        q_star = jnp.concatenate([q, r], axis=1)

    t = jnp.maximum(
        jnp.dot(q_star, fc1t_ref[...], preferred_element_type=jnp.float32)
        + fc1b_ref[...], 0.0)
    out_ref[...] = (jnp.dot(t, fc4t_ref[...], preferred_element_type=jnp.float32)
                    + fc4b_ref[...])


def _readout(h, bcol, wiht, whht, bih, bhh, fc1t, fc1b, fc4t, fc4b):
    return pl.pallas_call(
        _readout_body,
        out_shape=jax.ShapeDtypeStruct((B, NC), jnp.float32),
    )(h, bcol, wiht, whht, bih, bhh, fc1t, fc1b, fc4t, fc4b)


def _affine_from_stats(s, q, g, bb):
    mean = s / E
    var = q / E - mean * mean
    a = g * lax.rsqrt(var + 1e-5)
    return a, bb - mean * a


def kernel(x, edge_index, edge_attr, batch, be_W1, be_b1, be_g1, be_bb1,
           be_W2, be_b2, be_g2, be_bb2, mlp_W1, mlp_b1, mlp_g1, mlp_bb1,
           mlp_W2, mlp_b2, mlp_g2, mlp_bb2, eps, W_ih, W_hh, b_ih, b_hh,
           fc1_W, fc1_b, fc4_W, fc4_b):
    src = edge_index[0]
    dst = edge_index[1]

    w1t_all = jnp.transpose(be_W1, (0, 2, 1))
    w2t_all = jnp.transpose(be_W2, (0, 2, 1))
    mw1t_all = jnp.transpose(mlp_W1, (0, 2, 1))
    mw2t_all = jnp.transpose(mlp_W2, (0, 2, 1))

    s1, q1 = _edge_stats1(edge_attr, w1t_all)

    z2s, acs = [], []
    for l in range(L):
        a1, c1 = _affine_from_stats(s1[l], q1[l], be_g1[l], be_bb1[l])
        z2, s2, q2 = _edge_mlp(edge_attr, w1t_all[l], a1[None], c1[None],
                               w2t_all[l])
        a2, c2 = _affine_from_stats(s2[0], q2[0], be_g2[l], be_bb2[l])
        z2s.append(z2)
        acs.append(jnp.stack([a2, c2]))

    h = x
    for l in range(L):
        agg = _sc_msg(h, z2s[l], src, dst, acs[l])
        h = _node_mlp(h, agg, eps[l].reshape(1, 1), mw1t_all[l],
                      mlp_g1[l][None], mlp_bb1[l][None], mw2t_all[l],
                      mlp_g2[l][None], mlp_bb2[l][None])

    out = _readout(h, batch.reshape(N, 1), W_ih.T, W_hh.T, b_ih[None],
                   b_hh[None], fc1_W.T, fc1_b[None], fc4_W.T, fc4_b[None])
    return out
